# baseline (speedup 1.0000x reference)
import functools

import jax
import jax.numpy as jnp
from jax import lax
from jax.experimental import pallas as pl
from jax.experimental.pallas import tpu as pltpu
from jax.experimental.pallas import tpu_sc as plsc

B = 16384
D = 64
NV = 1000000
NUM_CORES = 2
NUM_SUBCORES = 16
NW = NUM_CORES * NUM_SUBCORES
BPW = B // NW
L = 16
KB = 8
NKB = D // KB


def _accum_block(ubufT, ibufT, sc_v, kb, first):
    k0 = kb * KB

    def grp(g, carry):
        sl = pl.ds(g * L, L)
        acc = jnp.zeros((L,), jnp.float32) if first else sc_v[sl]
        for kk in range(KB):
            acc = acc + ubufT[k0 + kk, sl] * ibufT[k0 + kk, sl]
        sc_v[sl] = acc
        return carry

    lax.fori_loop(0, BPW // L, grp, 0)


def _body(uids_hbm, iids_hbm, utabT_hbm, itabT_hbm,
          score_hbm, uoutT_hbm, ioutT_hbm,
          uidx_v, iidx_v, ubufT, ibufT, sc_v,
          sem_u, sem_i, sem_o):
    wid = lax.axis_index("s") * NUM_CORES + lax.axis_index("c")
    base = wid * BPW

    pltpu.sync_copy(uids_hbm.at[pl.ds(base, BPW)], uidx_v)
    pltpu.sync_copy(iids_hbm.at[pl.ds(base, BPW)], iidx_v)

    def fire(kb):
        copies = []
        k0 = kb * KB
        for kk in range(KB):
            copies.append(pltpu.async_copy(
                utabT_hbm.at[k0 + kk].at[uidx_v], ubufT.at[k0 + kk], sem_u))
            copies.append(pltpu.async_copy(
                itabT_hbm.at[k0 + kk].at[iidx_v], ibufT.at[k0 + kk], sem_i))
        return copies

    pend = fire(0)
    for kb in range(NKB):
        for cp in pend:
            cp.wait()
        if kb + 1 < NKB:
            pend = fire(kb + 1)
        _accum_block(ubufT, ibufT, sc_v, kb, first=(kb == 0))

    pltpu.sync_copy(sc_v, score_hbm.at[pl.ds(base, BPW)])
    co_u = pltpu.async_copy(ubufT, uoutT_hbm.at[:, pl.ds(base, BPW)], sem_o)
    co_i = pltpu.async_copy(ibufT, ioutT_hbm.at[:, pl.ds(base, BPW)], sem_o)
    co_u.wait()
    co_i.wait()


@jax.jit
def _run(user_ids, item_ids, user_table, item_table):
    utab_t = user_table[:NV].T
    itab_t = item_table[:NV].T
    mesh = plsc.VectorSubcoreMesh(core_axis_name="c", subcore_axis_name="s")
    kern = functools.partial(
        pl.kernel,
        out_type=[
            jax.ShapeDtypeStruct((B,), jnp.float32),
            jax.ShapeDtypeStruct((D, B), jnp.float32),
            jax.ShapeDtypeStruct((D, B), jnp.float32),
        ],
        mesh=mesh,
        compiler_params=pltpu.CompilerParams(
            needs_layout_passes=False, use_tc_tiling_on_sc=False),
        scratch_types=[
            pltpu.VMEM((BPW,), jnp.int32),
            pltpu.VMEM((BPW,), jnp.int32),
            pltpu.VMEM((D, BPW), jnp.float32),
            pltpu.VMEM((D, BPW), jnp.float32),
            pltpu.VMEM((BPW,), jnp.float32),
            pltpu.SemaphoreType.DMA,
            pltpu.SemaphoreType.DMA,
            pltpu.SemaphoreType.DMA,
        ],
    )(_body)
    score, u_embT, i_embT = kern(user_ids, item_ids, utab_t, itab_t)
    return score, u_embT.T, i_embT.T


def kernel(user_ids, item_ids, user_table, item_table):
    score, u_emb, i_emb = _run(
        user_ids.astype(jnp.int32), item_ids.astype(jnp.int32),
        user_table, item_table)
    b = user_ids.shape[0]
    return (score, u_emb.reshape(b, 1, D), i_emb.reshape(b, D, 1))


# speedup vs baseline: 9.5369x; 9.5369x over previous
import functools

import jax
import jax.numpy as jnp
from jax import lax
from jax.experimental import pallas as pl
from jax.experimental.pallas import tpu as pltpu
from jax.experimental.pallas import tpu_sc as plsc

B = 16384
D = 64
NV = 1000000
NUM_CORES = 2
NUM_SUBCORES = 16
NW = NUM_CORES * NUM_SUBCORES
BPW = B // NW
L = 16
CHUNK = 128
NCH = BPW // CHUNK


def _dot_chunk(ubuf, ibuf, uidx_v, iidx_v, sc_v, c):
    lane = lax.iota(jnp.int32, L)

    def grp(g, carry):
        r0 = g * L
        rows = lane + r0
        acc = jnp.zeros((L,), jnp.float32)
        for k in range(D):
            cols = jnp.full((L,), k, jnp.int32)
            u = plsc.load_gather(ubuf, [rows, cols])
            v = plsc.load_gather(ibuf, [rows, cols])
            acc = acc + u * v
        sc_v[pl.ds(c * CHUNK + r0, L)] = acc
        return carry

    lax.fori_loop(0, CHUNK // L, grp, 0)


def _body(uids_hbm, iids_hbm, utab2_hbm, itab2_hbm,
          score_hbm, uout2_hbm, iout2_hbm,
          uidx_v, iidx_v, ubuf0, ibuf0, ubuf1, ibuf1, sc_v,
          sem_g0, sem_g1, sem_o):
    wid = lax.axis_index("s") * NUM_CORES + lax.axis_index("c")
    base = wid * BPW

    pltpu.sync_copy(uids_hbm.at[pl.ds(base, BPW)], uidx_v)
    pltpu.sync_copy(iids_hbm.at[pl.ds(base, BPW)], iidx_v)

    ubufs = (ubuf0, ubuf1)
    ibufs = (ibuf0, ibuf1)
    gsems = (sem_g0, sem_g1)

    def start_gather(c):
        slot = c % 2
        sl = pl.ds(c * CHUNK, CHUNK)
        cu = pltpu.async_copy(utab2_hbm.at[uidx_v.at[sl]], ubufs[slot], gsems[slot])
        ci = pltpu.async_copy(itab2_hbm.at[iidx_v.at[sl]], ibufs[slot], gsems[slot])
        return cu, ci

    pend = start_gather(0)
    for c in range(NCH):
        slot = c % 2
        cu, ci = pend
        cu.wait()
        ci.wait()
        if c + 1 < NCH:
            pend = start_gather(c + 1)
        hb = base + c * CHUNK
        co_u = pltpu.async_copy(ubufs[slot], uout2_hbm.at[pl.ds(hb, CHUNK)], sem_o)
        co_i = pltpu.async_copy(ibufs[slot], iout2_hbm.at[pl.ds(hb, CHUNK)], sem_o)
        _dot_chunk(ubufs[slot], ibufs[slot], uidx_v, iidx_v, sc_v, c)
        co_u.wait()
        co_i.wait()

    pltpu.sync_copy(sc_v, score_hbm.at[pl.ds(base, BPW)])


@jax.jit
def _run(user_ids, item_ids, user_table, item_table):
    utab2 = jnp.pad(user_table[:NV], ((0, 0), (0, D)))
    itab2 = jnp.pad(item_table[:NV], ((0, 0), (0, D)))
    mesh = plsc.VectorSubcoreMesh(core_axis_name="c", subcore_axis_name="s")
    kern = functools.partial(
        pl.kernel,
        out_type=[
            jax.ShapeDtypeStruct((B,), jnp.float32),
            jax.ShapeDtypeStruct((B, 2 * D), jnp.float32),
            jax.ShapeDtypeStruct((B, 2 * D), jnp.float32),
        ],
        mesh=mesh,
        compiler_params=pltpu.CompilerParams(
            needs_layout_passes=False, use_tc_tiling_on_sc=False),
        scratch_types=[
            pltpu.VMEM((BPW,), jnp.int32),
            pltpu.VMEM((BPW,), jnp.int32),
            pltpu.VMEM((CHUNK, 2 * D), jnp.float32),
            pltpu.VMEM((CHUNK, 2 * D), jnp.float32),
            pltpu.VMEM((CHUNK, 2 * D), jnp.float32),
            pltpu.VMEM((CHUNK, 2 * D), jnp.float32),
            pltpu.VMEM((BPW,), jnp.float32),
            pltpu.SemaphoreType.DMA,
            pltpu.SemaphoreType.DMA,
            pltpu.SemaphoreType.DMA,
        ],
    )(_body)
    score, u_pad, i_pad = kern(user_ids, item_ids, utab2, itab2)
    return score, u_pad[:, :D], i_pad[:, :D]


def kernel(user_ids, item_ids, user_table, item_table):
    score, u_emb, i_emb = _run(
        user_ids.astype(jnp.int32), item_ids.astype(jnp.int32),
        user_table, item_table)
    b = user_ids.shape[0]
    return (score, u_emb.reshape(b, 1, D), i_emb.reshape(b, D, 1))


# speedup vs baseline: 9.6443x; 1.0113x over previous
import functools

import jax
import jax.numpy as jnp
from jax import lax
from jax.experimental import pallas as pl
from jax.experimental.pallas import tpu as pltpu
from jax.experimental.pallas import tpu_sc as plsc

B = 16384
D = 64
NV = 1000000
NUM_CORES = 2
NUM_SUBCORES = 16
NW = NUM_CORES * NUM_SUBCORES
BPW = B // NW
CHUNK = 128
NCH = BPW // CHUNK
TC_BLK = 2048


def _gather_body(uids_hbm, iids_hbm, utab2_hbm, itab2_hbm,
                 uout2_hbm, iout2_hbm,
                 uidx_v, iidx_v, ubuf0, ibuf0, ubuf1, ibuf1,
                 sem_g0, sem_g1, sem_o):
    wid = lax.axis_index("s") * NUM_CORES + lax.axis_index("c")
    base = wid * BPW

    pltpu.sync_copy(uids_hbm.at[pl.ds(base, BPW)], uidx_v)
    pltpu.sync_copy(iids_hbm.at[pl.ds(base, BPW)], iidx_v)

    ubufs = (ubuf0, ubuf1)
    ibufs = (ibuf0, ibuf1)
    gsems = (sem_g0, sem_g1)

    def start_gather(c):
        slot = c % 2
        sl = pl.ds(c * CHUNK, CHUNK)
        cu = pltpu.async_copy(utab2_hbm.at[uidx_v.at[sl]], ubufs[slot], gsems[slot])
        ci = pltpu.async_copy(itab2_hbm.at[iidx_v.at[sl]], ibufs[slot], gsems[slot])
        return cu, ci

    pend = start_gather(0)
    prev_out = None
    for c in range(NCH):
        slot = c % 2
        cu, ci = pend
        cu.wait()
        ci.wait()
        if c + 1 < NCH:
            pend = start_gather(c + 1)
        if prev_out is not None:
            for cp in prev_out:
                cp.wait()
        hb = base + c * CHUNK
        prev_out = (
            pltpu.async_copy(ubufs[slot], uout2_hbm.at[pl.ds(hb, CHUNK)], sem_o),
            pltpu.async_copy(ibufs[slot], iout2_hbm.at[pl.ds(hb, CHUNK)], sem_o),
        )
    for cp in prev_out:
        cp.wait()


def _dot_body(u_ref, i_ref, o_ref):
    prod = u_ref[...] * i_ref[...]
    o_ref[...] = jnp.sum(prod[:, :D], axis=1)


@jax.jit
def _run(user_ids, item_ids, user_table, item_table):
    utab2 = jnp.pad(user_table[:NV], ((0, 0), (0, D)))
    itab2 = jnp.pad(item_table[:NV], ((0, 0), (0, D)))
    mesh = plsc.VectorSubcoreMesh(core_axis_name="c", subcore_axis_name="s")
    gather = functools.partial(
        pl.kernel,
        out_type=[
            jax.ShapeDtypeStruct((B, 2 * D), jnp.float32),
            jax.ShapeDtypeStruct((B, 2 * D), jnp.float32),
        ],
        mesh=mesh,
        compiler_params=pltpu.CompilerParams(needs_layout_passes=False),
        scratch_types=[
            pltpu.VMEM((BPW,), jnp.int32),
            pltpu.VMEM((BPW,), jnp.int32),
            pltpu.VMEM((CHUNK, 2 * D), jnp.float32),
            pltpu.VMEM((CHUNK, 2 * D), jnp.float32),
            pltpu.VMEM((CHUNK, 2 * D), jnp.float32),
            pltpu.VMEM((CHUNK, 2 * D), jnp.float32),
            pltpu.SemaphoreType.DMA,
            pltpu.SemaphoreType.DMA,
            pltpu.SemaphoreType.DMA,
        ],
    )(_gather_body)
    u_pad, i_pad = gather(user_ids, item_ids, utab2, itab2)

    score = pl.pallas_call(
        _dot_body,
        grid=(B // TC_BLK,),
        in_specs=[
            pl.BlockSpec((TC_BLK, 2 * D), lambda g: (g, 0)),
            pl.BlockSpec((TC_BLK, 2 * D), lambda g: (g, 0)),
        ],
        out_specs=pl.BlockSpec((TC_BLK,), lambda g: (g,)),
        out_shape=jax.ShapeDtypeStruct((B,), jnp.float32),
    )(u_pad, i_pad)

    return score, u_pad[:, :D], i_pad[:, :D]


def kernel(user_ids, item_ids, user_table, item_table):
    score, u_emb, i_emb = _run(
        user_ids.astype(jnp.int32), item_ids.astype(jnp.int32),
        user_table, item_table)
    b = user_ids.shape[0]
    return (score, u_emb.reshape(b, 1, D), i_emb.reshape(b, D, 1))
